# baseline (speedup 1.0000x reference)
import functools
import math

import jax
import jax.numpy as jnp
from jax.experimental import pallas as pl
from jax.experimental.pallas import tpu as pltpu

_NUM_CLASSES = 80
_STRIDE = 8.0
_GRID = 80
_A = 3
_T = 20
_AW = (10.0, 16.0, 33.0)
_AH = (13.0, 30.0, 23.0)


def _bce(x, t):
    return jnp.maximum(x, 0.0) - x * t + jnp.log1p(jnp.exp(-jnp.abs(x)))


def _atan_pos(x):
    inv = x > 1.0
    z = jnp.where(inv, 1.0 / x, x)
    z2 = z * z
    p = -0.0117212
    p = p * z2 + 0.05265332
    p = p * z2 - 0.11643287
    p = p * z2 + 0.19354346
    p = p * z2 - 0.33262347
    p = p * z2 + 0.99997726
    p = p * z
    return jnp.where(inv, (math.pi / 2) - p, p)


def _loss_body(pred_ref, boxes_s, boxes_col_ref, boxes_row_ref,
               lab_col_ref, lab_row_ref, out_ref):
    idx = pl.program_id(0)
    b = idx // _A
    a = idx % _A
    af = a.astype(jnp.float32)
    aw = jnp.where(a == 0, _AW[0], jnp.where(a == 1, _AW[1], _AW[2]))
    ah = jnp.where(a == 0, _AH[0], jnp.where(a == 1, _AH[1], _AH[2]))

    ri = jax.lax.broadcasted_iota(jnp.int32, (_GRID, _GRID), 0)
    ci = jax.lax.broadcasted_iota(jnp.int32, (_GRID, _GRID), 1)

    px = pred_ref[0, :, :, 0]
    py = pred_ref[0, :, :, 1]
    pw = pred_ref[0, :, :, 2]
    ph = pred_ref[0, :, :, 3]
    po = pred_ref[0, :, :, 4]
    sx = 1.0 / (1.0 + jnp.exp(-px))
    sy = 1.0 / (1.0 + jnp.exp(-py))
    cxp = (sx + ci.astype(jnp.float32)) * _STRIDE
    cyp = (sy + ri.astype(jnp.float32)) * _STRIDE
    wp = aw * jnp.exp(jnp.clip(pw, -10.0, 10.0))
    hp = ah * jnp.exp(jnp.clip(ph, -10.0, 10.0))
    x1p = cxp - wp * 0.5
    y1p = cyp - hp * 0.5
    x2p = cxp + wp * 0.5
    y2p = cyp + hp * 0.5
    area_p = (x2p - x1p) * (y2p - y1p)

    def _t_step(t, carry):
        obj, hi, tx1, ty1, tx2, ty2 = carry
        x1 = boxes_s[b, t, 0]
        y1 = boxes_s[b, t, 1]
        x2 = boxes_s[b, t, 2]
        y2 = boxes_s[b, t, 3]
        cx = (x1 + x2) * 0.5
        cy = (y1 + y2) * 0.5
        w = x2 - x1
        h = y2 - y1
        garea = w * h
        iou_a = []
        for k in range(3):
            inter = jnp.minimum(w, _AW[k]) * jnp.minimum(h, _AH[k])
            iou_a.append(inter / (_AW[k] * _AH[k] + garea - inter + 1e-6))
        b1 = iou_a[1] > iou_a[0]
        bv = jnp.where(b1, iou_a[1], iou_a[0])
        besta = jnp.where(b1, 1, 0)
        besta = jnp.where(iou_a[2] > bv, 2, besta)
        gi = jnp.clip((cx / _STRIDE).astype(jnp.int32), 0, _GRID - 1)
        gj = jnp.clip((cy / _STRIDE).astype(jnp.int32), 0, _GRID - 1)
        valid = besta == a
        mask = jnp.logical_and(jnp.logical_and(ri == gj, ci == gi), valid)
        maskf = mask.astype(jnp.float32)
        obj = jnp.maximum(obj, maskf)
        tx1 = jnp.where(mask, x1, tx1)
        ty1 = jnp.where(mask, y1, ty1)
        tx2 = jnp.where(mask, x2, tx2)
        ty2 = jnp.where(mask, y2, ty2)
        xx1 = jnp.maximum(x1, x1p)
        yy1 = jnp.maximum(y1, y1p)
        xx2 = jnp.minimum(x2, x2p)
        yy2 = jnp.minimum(y2, y2p)
        inter = jnp.clip(xx2 - xx1, 0.0) * jnp.clip(yy2 - yy1, 0.0)
        iou = inter / (garea + area_p - inter + 1e-6)
        hi = jnp.maximum(hi, (iou > 0.5).astype(jnp.float32))
        return (obj, hi, tx1, ty1, tx2, ty2)

    zero = jnp.zeros((_GRID, _GRID), dtype=jnp.float32)
    init = (zero, zero, zero, zero,
            jnp.full((_GRID, _GRID), 10.0, dtype=jnp.float32),
            jnp.full((_GRID, _GRID), 10.0, dtype=jnp.float32))
    objf, hif, tx1, ty1, tx2, ty2 = jax.lax.fori_loop(0, _T, _t_step, init)

    obj = objf > 0.0
    noobjf = (1.0 - objf) * (1.0 - hif)
    n_pos = jnp.sum(objf)
    n_no = jnp.sum(noobjf)

    bce_o = _bce(po, objf)
    s_obj_pos = jnp.sum(bce_o * objf)
    s_obj_no = jnp.sum(bce_o * noobjf)

    ix1 = jnp.maximum(x1p, tx1)
    iy1 = jnp.maximum(y1p, ty1)
    ix2 = jnp.minimum(x2p, tx2)
    iy2 = jnp.minimum(y2p, ty2)
    inter = jnp.clip(ix2 - ix1, 0.0) * jnp.clip(iy2 - iy1, 0.0)
    area_t = (tx2 - tx1) * (ty2 - ty1)
    iou = inter / (area_p + area_t - inter + 1e-6)
    tw = tx2 - tx1
    th = ty2 - ty1
    pcx = (x1p + x2p) * 0.5
    pcy = (y1p + y2p) * 0.5
    tcx = (tx1 + tx2) * 0.5
    tcy = (ty1 + ty2) * 0.5
    cw = jnp.maximum(x2p, tx2) - jnp.minimum(x1p, tx1)
    ch = jnp.maximum(y2p, ty2) - jnp.minimum(y1p, ty1)
    c2 = cw * cw + ch * ch + 1e-7
    rho2 = (pcx - tcx) ** 2 + (pcy - tcy) ** 2
    v = (4.0 / (math.pi ** 2)) * (_atan_pos(tw / (th + 1e-7)) - _atan_pos(wp / (hp + 1e-7))) ** 2
    alpha = v / (1.0 - iou + v + 1e-7)
    ciou_l = 1.0 - (iou - rho2 / c2 - alpha * v)
    s_ciou = jnp.sum(ciou_l * objf)

    bc = boxes_col_ref[0]
    cx_c = (bc[:, 0:1] + bc[:, 2:3]) * 0.5
    cy_c = (bc[:, 1:2] + bc[:, 3:4]) * 0.5
    w_c = bc[:, 2:3] - bc[:, 0:1]
    h_c = bc[:, 3:4] - bc[:, 1:2]
    br = boxes_row_ref[0]
    cx_r = (br[0:1, :] + br[2:3, :]) * 0.5
    cy_r = (br[1:2, :] + br[3:4, :]) * 0.5
    w_r = br[2:3, :] - br[0:1, :]
    h_r = br[3:4, :] - br[1:2, :]

    def _besta(w_, h_):
        ga = w_ * h_
        ious = []
        for k in range(3):
            it = jnp.minimum(w_, _AW[k]) * jnp.minimum(h_, _AH[k])
            ious.append(it / (_AW[k] * _AH[k] + ga - it + 1e-6))
        b1_ = ious[1] > ious[0]
        bv_ = jnp.where(b1_, ious[1], ious[0])
        bb = jnp.where(b1_, 1.0, 0.0)
        return jnp.where(ious[2] > bv_, 2.0, bb)

    besta_c = _besta(w_c, h_c)
    besta_r = _besta(w_r, h_r)
    gi_c = jnp.clip((cx_c / _STRIDE).astype(jnp.int32), 0, _GRID - 1)
    gj_c = jnp.clip((cy_c / _STRIDE).astype(jnp.int32), 0, _GRID - 1)
    gi_r = jnp.clip((cx_r / _STRIDE).astype(jnp.int32), 0, _GRID - 1)
    gj_r = jnp.clip((cy_r / _STRIDE).astype(jnp.int32), 0, _GRID - 1)
    valid_c = besta_c == af
    valid_r = besta_r == af
    lab_c = lab_col_ref[0]
    lab_r = lab_row_ref[0]

    t_c = jax.lax.broadcasted_iota(jnp.int32, (_T, _T), 0)
    t_r = jax.lax.broadcasted_iota(jnp.int32, (_T, _T), 1)
    earlier = t_r < t_c
    same_cell = jnp.logical_and(gi_c == gi_r, gj_c == gj_r)
    prev_valid = jnp.logical_and(earlier, valid_r)
    dup_cell = jnp.any(jnp.logical_and(prev_valid, same_cell), axis=1, keepdims=True)
    same_lab = jnp.logical_and(same_cell, lab_c == lab_r)
    dup_lab = jnp.any(jnp.logical_and(prev_valid, same_lab), axis=1, keepdims=True)
    first_c = jnp.logical_and(valid_c, jnp.logical_not(dup_cell)).astype(jnp.float32)
    sub_c = jnp.logical_and(valid_c, jnp.logical_not(dup_lab)).astype(jnp.float32)

    cr = jax.lax.broadcasted_iota(jnp.int32, (_T, _GRID * _GRID), 1)
    sel = jnp.logical_and(jnp.logical_and((cr // _GRID) == gj_c, (cr % _GRID) == gi_c),
                          valid_c).astype(jnp.float32)
    xcls = pred_ref[0, :, :, 5:5 + _NUM_CLASSES].reshape(_GRID * _GRID, _NUM_CLASSES)
    rows = jax.lax.dot_general(sel, xcls, (((1,), (0,)), ((), ())),
                               preferred_element_type=jnp.float32)
    bce0 = jnp.maximum(rows, 0.0) + jnp.log1p(jnp.exp(-jnp.abs(rows)))
    s0 = jnp.sum(bce0, axis=1, keepdims=True)
    onehot = (jax.lax.broadcasted_iota(jnp.int32, (_T, _NUM_CLASSES), 1) == lab_c)
    xl = jnp.sum(rows * onehot.astype(jnp.float32), axis=1, keepdims=True)
    s_cls = jnp.sum(first_c * s0) - jnp.sum(sub_c * xl)

    lane = jax.lax.broadcasted_iota(jnp.int32, (1, 128), 1)
    vals = (n_pos * (lane == 0) + s_ciou * (lane == 1) + s_obj_pos * (lane == 2)
            + s_obj_no * (lane == 3) + n_no * (lane == 4) + s_cls * (lane == 5))
    out_ref[0, :, :] = vals.astype(jnp.float32)


@jax.jit
def kernel(pred, boxes, labels):
    B, A, H, W, C = pred.shape
    pred_r = pred.reshape(B * A, H, W, C)
    boxes_row = jnp.transpose(boxes, (0, 2, 1))
    labels = labels.astype(jnp.int32)
    lab_col = labels[:, :, None]
    lab_row = labels[:, None, :]
    parts = pl.pallas_call(
        _loss_body,
        grid=(B * A,),
        in_specs=[
            pl.BlockSpec((1, H, W, C), lambda i: (i, 0, 0, 0)),
            pl.BlockSpec(memory_space=pltpu.SMEM),
            pl.BlockSpec((1, _T, 4), lambda i: (i // _A, 0, 0)),
            pl.BlockSpec((1, 4, _T), lambda i: (i // _A, 0, 0)),
            pl.BlockSpec((1, _T, 1), lambda i: (i // _A, 0, 0)),
            pl.BlockSpec((1, 1, _T), lambda i: (i // _A, 0, 0)),
        ],
        out_specs=pl.BlockSpec((1, 1, 128), lambda i: (i, 0, 0)),
        out_shape=jax.ShapeDtypeStruct((B * A, 1, 128), jnp.float32),
    )(pred_r, boxes, boxes, boxes_row, lab_col, lab_row)
    p = parts.sum(axis=(0, 1))
    n_pos, s_ciou, s_op, s_on, n_no, s_cls = p[0], p[1], p[2], p[3], p[4], p[5]
    loss_box = s_ciou / jnp.maximum(n_pos, 1.0)
    loss_obj = 1.0 * s_op / (n_pos + 1.0) + 0.5 * s_on / (n_no + 1.0)
    loss_cls = s_cls / (n_pos + 1.0)
    total = 0.05 * loss_box + loss_obj + 0.5 * loss_cls
    return (total, loss_box, loss_obj, loss_cls)
